# Initial kernel scaffold; baseline (speedup 1.0000x reference)
import jax
import jax.numpy as jnp
from jax.experimental import pallas as pl


def kernel(x_tangent0, edge_index, edge_weight, W):
    raise NotImplementedError("write your pallas kernel here")


# speedup vs baseline: 53.7793x; 53.7793x over previous
import functools

import jax
import jax.numpy as jnp
from jax import lax
from jax.experimental import pallas as pl
from jax.experimental.pallas import tpu as pltpu
from jax.experimental.pallas import tpu_sc as plsc

N_NODES = 10000
N_EDGES = 320000
D = 128
HEADS = 4
C = 32
AW = 16

NUM_CORES = 2
NUM_TILES = 16
NW = NUM_CORES * NUM_TILES
E_PER_W = N_EDGES // NW
BLK = 80
N_BLKS = E_PER_W // BLK
N_PAD = 10240
ROWS_PER_TILE = N_PAD // NUM_TILES


def _prep_body(x_ref, w_ref, o_ref):
    o_ref[...] = lax.dot_general(
        x_ref[...], w_ref[...], (((1,), (0,)), ((), ())),
        preferred_element_type=jnp.float32, precision=lax.Precision.HIGHEST)


_tc_prep = pl.pallas_call(
    _prep_body,
    out_shape=jax.ShapeDtypeStruct((N_NODES, AW), jnp.float32),
)


def _comb_body(p_ref, o_ref):
    o_ref[...] = p_ref[0, :N_NODES] + p_ref[1, :N_NODES]


_tc_combine = pl.pallas_call(
    _comb_body,
    out_shape=jax.ShapeDtypeStruct((N_NODES, D), jnp.float32),
)


def _sc_body(a_hbm, src_hbm, dst_hbm, ew_hbm, x_hbm, zeros_hbm, out_hbm,
             src_v, dst_v, ew_v, ai_v, aj_v, betat_v, xj_v, shared_out,
             sem_a, sem_b, sem_x):
    c = lax.axis_index("c")
    s = lax.axis_index("s")
    tid = c * NUM_TILES + s

    pltpu.sync_copy(zeros_hbm, shared_out.at[pl.ds(s * ROWS_PER_TILE, ROWS_PER_TILE)])
    plsc.subcore_barrier()

    edge0 = tid * E_PER_W

    def block(i, carry):
        base = edge0 + i * BLK
        pltpu.sync_copy(src_hbm.at[pl.ds(base, BLK)], src_v)
        pltpu.sync_copy(dst_hbm.at[pl.ds(base, BLK)], dst_v)
        pltpu.sync_copy(ew_hbm.at[pl.ds(base, BLK)], ew_v)
        ca = pltpu.async_copy(a_hbm.at[src_v], ai_v, sem_a)
        cb = pltpu.async_copy(a_hbm.at[dst_v], aj_v, sem_b)
        cx = pltpu.async_copy(x_hbm.at[dst_v], xj_v, sem_x)
        ca.wait()
        cb.wait()

        for g in range(BLK // 16):
            sl = pl.ds(g * 16, 16)
            rows = jnp.full((16,), g * 16, jnp.int32) + lax.iota(jnp.int32, 16)
            w16 = ew_v[sl]
            for h in range(HEADS):
                a1 = plsc.load_gather(ai_v, [rows, jnp.full((16,), h, jnp.int32)])
                a2 = plsc.load_gather(aj_v, [rows, jnp.full((16,), HEADS + h, jnp.int32)])
                beta = w16 / (1.0 + jnp.exp(-(a1 + a2)))
                betat_v[pl.ds(h * BLK + g * 16, 16)] = beta

        cx.wait()

        def edge(e, carry2):
            e_idx = jnp.full((16,), e, jnp.int32)
            for h in range(HEADS):
                b = plsc.load_gather(betat_v, [e_idx + (h * BLK)])
                for k in range(C // 16):
                    fsl = pl.ds(h * C + k * 16, 16)
                    xj_v[e, fsl] = xj_v[e, fsl] * b
            return carry2

        lax.fori_loop(0, BLK, edge, 0)

        pltpu.sync_copy(xj_v, shared_out.at[src_v], add=True)
        return carry

    lax.fori_loop(0, N_BLKS, block, 0)
    plsc.subcore_barrier()

    rsl = pl.ds(s * ROWS_PER_TILE, ROWS_PER_TILE)
    pltpu.sync_copy(shared_out.at[rsl], out_hbm.at[c, rsl])


_sc_main = functools.partial(
    pl.kernel,
    out_type=jax.ShapeDtypeStruct((NUM_CORES, N_PAD, D), jnp.float32),
    mesh=plsc.VectorSubcoreMesh(core_axis_name="c", subcore_axis_name="s"),
    compiler_params=pltpu.CompilerParams(
        needs_layout_passes=False, use_tc_tiling_on_sc=False),
    scratch_types=[
        pltpu.VMEM((BLK,), jnp.int32),
        pltpu.VMEM((BLK,), jnp.int32),
        pltpu.VMEM((BLK,), jnp.float32),
        pltpu.VMEM((BLK, AW), jnp.float32),
        pltpu.VMEM((BLK, AW), jnp.float32),
        pltpu.VMEM((HEADS * BLK,), jnp.float32),
        pltpu.VMEM((BLK, D), jnp.float32),
        pltpu.VMEM_SHARED((N_PAD, D), jnp.float32),
        pltpu.SemaphoreType.DMA,
        pltpu.SemaphoreType.DMA,
        pltpu.SemaphoreType.DMA,
    ],
)(_sc_body)


def kernel(x_tangent0, edge_index, edge_weight, W):
    src = edge_index[0].astype(jnp.int32)
    dst = edge_index[1].astype(jnp.int32)
    w1 = W[0, :C]
    w2 = W[0, C:]
    eye = jnp.eye(HEADS, dtype=jnp.float32)
    wm = jnp.concatenate(
        [jnp.kron(eye, w1[:, None]), jnp.kron(eye, w2[:, None]),
         jnp.zeros((D, AW - 2 * HEADS), jnp.float32)], axis=1)
    a = _tc_prep(x_tangent0, wm)
    zeros = jnp.zeros((ROWS_PER_TILE, D), jnp.float32)
    partials = _sc_main(a, src, dst, edge_weight, x_tangent0, zeros)
    return _tc_combine(partials)
